# baseline (speedup 1.0000x reference)
import functools

import jax
import jax.numpy as jnp
from jax import lax
from jax.experimental import pallas as pl
from jax.experimental.pallas import tpu as pltpu
from jax.experimental.pallas import tpu_sc as plsc

E = 320000
N = 10000
D = 128
H = 32
CO = 65

NC = 2
NS = 16
NW = NC * NS
EPW = E // NW

CH = 40
OCH = CH // 2
NROWS = EPW // CH
NBUF = 5
NT = NROWS // NBUF


def _gather_add_body(src_hbm, dst_hbm, table_hbm, out_hbm,
                     ibuf,
                     ab0, ab1, ab2, ab3, ab4, bb0, bb1, bb2, bb3, bb4,
                     sb0, sb1, sb2, sb3, sb4,
                     g0, g1, g2, g3, g4, s0, s1, s2, s3, s4):
    abufs = (ab0, ab1, ab2, ab3, ab4)
    bbufs = (bb0, bb1, bb2, bb3, bb4)
    sbufs = (sb0, sb1, sb2, sb3, sb4)
    gsems = (g0, g1, g2, g3, g4)
    ssems = (s0, s1, s2, s3, s4)
    wid = lax.axis_index("s") * NC + lax.axis_index("c")
    obase = wid * (EPW // 2)

    def fetch_idx(t):
        pltpu.sync_copy(src_hbm.at[wid, t], ibuf.at[0])
        pltpu.sync_copy(dst_hbm.at[wid, t], ibuf.at[1])

    def issue(b):
        pltpu.async_copy(table_hbm.at[ibuf.at[0, b]], abufs[b], gsems[b])
        pltpu.async_copy(table_hbm.at[ibuf.at[1, b]], bbufs[b], gsems[b])

    def wait_gather(b):
        pltpu.make_async_copy(
            table_hbm.at[ibuf.at[0, b]], abufs[b], gsems[b]).wait()
        pltpu.make_async_copy(
            table_hbm.at[ibuf.at[1, b]], bbufs[b], gsems[b]).wait()

    def wait_store(cc, b):
        pltpu.make_async_copy(
            sbufs[b], out_hbm.at[pl.ds(obase + cc * OCH, OCH)], ssems[b]).wait()

    fetch_idx(0)
    for b in range(NBUF):
        issue(b)

    def outer(t, carry):
        c0 = t * NBUF
        for b in range(NBUF):
            cc = c0 + b
            wait_gather(b)
            a, bb, sb = abufs[b], bbufs[b], sbufs[b]

            def add_row(r2, carry2, a=a, bb=bb, sb=sb):
                r0 = r2 * 2
                r1 = r0 + 1
                for q in range(D // 16):
                    slc = pl.ds(q * 16, 16)
                    s0v = a[r0, slc] + bb[r0, slc]
                    s1v = a[r1, slc] + bb[r1, slc]
                    v = jnp.stack([s0v.astype(jnp.bfloat16),
                                   s1v.astype(jnp.bfloat16)], axis=0)
                    sb[r2, :, slc] = v
                return carry2
            lax.fori_loop(0, CH // 2, add_row, 0, unroll=2)
            pltpu.async_copy(sb, out_hbm.at[pl.ds(obase + cc * OCH, OCH)],
                             ssems[b])

        @pl.when(t < NT - 1)
        def _prep():
            fetch_idx(t + 1)
            for b in range(NBUF):
                wait_store(c0 + b, b)
                issue(b)
        return carry

    lax.fori_loop(0, NT, outer, 0)
    for b in range(NBUF):
        wait_store(0, b)


@functools.lru_cache(maxsize=None)
def _make_gather_add():
    return pl.kernel(
        _gather_add_body,
        out_type=jax.ShapeDtypeStruct((E // 2, 2, D), jnp.bfloat16),
        mesh=plsc.VectorSubcoreMesh(core_axis_name="c", subcore_axis_name="s",
                                    num_cores=NC, num_subcores=NS),
        scratch_types=[
            pltpu.VMEM((2, NBUF, CH), jnp.int32),
        ] + [pltpu.VMEM((CH, D), jnp.float32) for _ in range(2 * NBUF)]
          + [pltpu.VMEM((OCH, 2, D), jnp.bfloat16) for _ in range(NBUF)]
          + [pltpu.SemaphoreType.DMA for _ in range(2 * NBUF)],
    )


import numpy as _np
_PACK_PERM = _np.array(
    [32 * g + (j // 2) + 16 * (j % 2) for g in range(4) for j in range(32)])

BLK = 2000


def _mlp_body(e_ref, w1_ref, b1_ref, w2_ref, b2_ref, o_ref):
    e = e_ref[...].astype(jnp.float32)
    h = jnp.dot(e, w1_ref[...], preferred_element_type=jnp.float32)
    h = jnp.maximum(h + b1_ref[...], 0.0)
    o = jnp.dot(h, w2_ref[...], preferred_element_type=jnp.float32)
    o = jnp.maximum(o + b2_ref[...], 0.0)
    m = jnp.max(o, axis=1, keepdims=True)
    p = jnp.exp(o - m)
    o_ref[...] = p / jnp.sum(p, axis=1, keepdims=True)


def _mlp(e, W1, b1, W2, b2):
    grid = (E // BLK,)
    return pl.pallas_call(
        _mlp_body,
        grid=grid,
        in_specs=[
            pl.BlockSpec((BLK, D), lambda i: (i, 0)),
            pl.BlockSpec((D, H), lambda i: (0, 0)),
            pl.BlockSpec((1, H), lambda i: (0, 0)),
            pl.BlockSpec((H, CO), lambda i: (0, 0)),
            pl.BlockSpec((1, CO), lambda i: (0, 0)),
        ],
        out_specs=pl.BlockSpec((BLK, CO), lambda i: (i, 0)),
        out_shape=jax.ShapeDtypeStruct((E, CO), jnp.float32),
    )(e, W1, b1, W2, b2)


def kernel(src, dst, table, W1, b1, W2, b2):
    src2d = src.reshape(NW, NT, NBUF, CH)
    dst2d = dst.reshape(NW, NT, NBUF, CH)
    e2 = _make_gather_add()(src2d, dst2d, table)
    e = e2.reshape(E, D)
    return _mlp(e, W1, b1.reshape(1, H), W2, b2.reshape(1, CO))


# speedup vs baseline: 1.1300x; 1.1300x over previous
import functools

import jax
import jax.numpy as jnp
from jax import lax
from jax.experimental import pallas as pl
from jax.experimental.pallas import tpu as pltpu
from jax.experimental.pallas import tpu_sc as plsc

E = 320000
N = 10000
D = 128
H = 32
CO = 65

NC = 2
NS = 16
NW = NC * NS

P = 2
EP = E // P
EPW = EP // NW
CH = 50
OCH = CH // 2
NROWS = EPW // CH
NBUF = 5
NT = NROWS // NBUF


def _gather_add_body(src_hbm, dst_hbm, table_hbm, out_hbm,
                     ibuf,
                     ab0, ab1, ab2, ab3, ab4, bb0, bb1, bb2, bb3, bb4,
                     sb0, sb1, sb2, sb3, sb4,
                     g0, g1, g2, g3, g4, s0, s1, s2, s3, s4):
    abufs = (ab0, ab1, ab2, ab3, ab4)
    bbufs = (bb0, bb1, bb2, bb3, bb4)
    sbufs = (sb0, sb1, sb2, sb3, sb4)
    gsems = (g0, g1, g2, g3, g4)
    ssems = (s0, s1, s2, s3, s4)
    wid = lax.axis_index("s") * NC + lax.axis_index("c")
    obase = wid * (EPW // 2)

    def fetch_idx(t):
        pltpu.sync_copy(src_hbm.at[wid, t], ibuf.at[0])
        pltpu.sync_copy(dst_hbm.at[wid, t], ibuf.at[1])

    def issue(b):
        pltpu.async_copy(table_hbm.at[ibuf.at[0, b]], abufs[b], gsems[b])
        pltpu.async_copy(table_hbm.at[ibuf.at[1, b]], bbufs[b], gsems[b])

    def wait_gather(b):
        pltpu.make_async_copy(
            table_hbm.at[ibuf.at[0, b]], abufs[b], gsems[b]).wait()
        pltpu.make_async_copy(
            table_hbm.at[ibuf.at[1, b]], bbufs[b], gsems[b]).wait()

    def wait_store(cc, b):
        pltpu.make_async_copy(
            sbufs[b], out_hbm.at[pl.ds(obase + cc * OCH, OCH)], ssems[b]).wait()

    fetch_idx(0)
    for b in range(NBUF):
        issue(b)

    def outer(t, carry):
        c0 = t * NBUF
        for b in range(NBUF):
            cc = c0 + b
            wait_gather(b)
            a, bb, sb = abufs[b], bbufs[b], sbufs[b]

            def add_row(r2, carry2, a=a, bb=bb, sb=sb):
                r0 = r2 * 2
                r1 = r0 + 1
                for q in range(D // 16):
                    slc = pl.ds(q * 16, 16)
                    s0v = a[r0, slc] + bb[r0, slc]
                    s1v = a[r1, slc] + bb[r1, slc]
                    v = jnp.stack([s0v.astype(jnp.bfloat16),
                                   s1v.astype(jnp.bfloat16)], axis=0)
                    sb[r2, :, slc] = v
                return carry2
            lax.fori_loop(0, CH // 2, add_row, 0, unroll=2)
            pltpu.async_copy(sb, out_hbm.at[pl.ds(obase + cc * OCH, OCH)],
                             ssems[b])

        @pl.when(t < NT - 1)
        def _prep():
            fetch_idx(t + 1)
            for b in range(NBUF):
                wait_store(c0 + b, b)
                issue(b)
        return carry

    lax.fori_loop(0, NT, outer, 0)
    for b in range(NBUF):
        wait_store(0, b)


@functools.lru_cache(maxsize=None)
def _make_gather_add():
    return pl.kernel(
        _gather_add_body,
        out_type=jax.ShapeDtypeStruct((EP // 2, 2, D), jnp.bfloat16),
        mesh=plsc.VectorSubcoreMesh(core_axis_name="c", subcore_axis_name="s",
                                    num_cores=NC, num_subcores=NS),
        scratch_types=[
            pltpu.VMEM((2, NBUF, CH), jnp.int32),
        ] + [pltpu.VMEM((CH, D), jnp.float32) for _ in range(2 * NBUF)]
          + [pltpu.VMEM((OCH, 2, D), jnp.bfloat16) for _ in range(NBUF)]
          + [pltpu.SemaphoreType.DMA for _ in range(2 * NBUF)],
    )


import numpy as _np
_PACK_PERM = _np.array(
    [32 * g + (j // 2) + 16 * (j % 2) for g in range(4) for j in range(32)])

BLK = 2000


def _mlp_body(e_ref, w1_ref, b1_ref, w2_ref, b2_ref, o_ref):
    e = e_ref[...].astype(jnp.float32)
    h = jnp.dot(e, w1_ref[...], preferred_element_type=jnp.float32)
    h = jnp.maximum(h + b1_ref[...], 0.0)
    o = jnp.dot(h, w2_ref[...], preferred_element_type=jnp.float32)
    o = jnp.maximum(o + b2_ref[...], 0.0)
    m = jnp.max(o, axis=1, keepdims=True)
    p = jnp.exp(o - m)
    o_ref[...] = p / jnp.sum(p, axis=1, keepdims=True)


def _mlp_acc_body(dest_ref, e_ref, w1_ref, b1_ref, w2_ref, b2_ref, o_ref):
    del dest_ref
    _mlp_body(e_ref, w1_ref, b1_ref, w2_ref, b2_ref, o_ref)


_NBLK = EP // BLK


def _mlp_phase(e, W1, b1, W2, b2, dest, block0):
    wspecs = [
        pl.BlockSpec((D, H), lambda i: (0, 0)),
        pl.BlockSpec((1, H), lambda i: (0, 0)),
        pl.BlockSpec((H, CO), lambda i: (0, 0)),
        pl.BlockSpec((1, CO), lambda i: (0, 0)),
    ]
    if dest is None:
        return pl.pallas_call(
            _mlp_body,
            grid=(_NBLK,),
            in_specs=[pl.BlockSpec((BLK, D), lambda i: (i, 0))] + wspecs,
            out_specs=pl.BlockSpec((BLK, CO), lambda i: (i, 0)),
            out_shape=jax.ShapeDtypeStruct((E, CO), jnp.float32),
        )(e, W1, b1, W2, b2)
    return pl.pallas_call(
        _mlp_acc_body,
        grid=(_NBLK,),
        in_specs=[pl.BlockSpec(memory_space=pl.ANY),
                  pl.BlockSpec((BLK, D), lambda i: (i, 0))] + wspecs,
        out_specs=pl.BlockSpec((BLK, CO), lambda i: (i + block0, 0)),
        out_shape=jax.ShapeDtypeStruct((E, CO), jnp.float32),
        input_output_aliases={0: 0},
    )(dest, e, W1, b1, W2, b2)


def kernel(src, dst, table, W1, b1, W2, b2):
    b1r = b1.reshape(1, H)
    b2r = b2.reshape(1, CO)
    gather = _make_gather_add()
    es = []
    for ph in range(P):
        s_p = lax.slice_in_dim(src, ph * EP, (ph + 1) * EP)
        d_p = lax.slice_in_dim(dst, ph * EP, (ph + 1) * EP)
        e2 = gather(s_p.reshape(NW, NT, NBUF, CH),
                    d_p.reshape(NW, NT, NBUF, CH), table)
        es.append(e2.reshape(EP, D))
    out = _mlp_phase(es[0], W1, b1r, W2, b2r, None, 0)
    for ph in range(1, P):
        out = _mlp_phase(es[ph], W1, b1r, W2, b2r, out, ph * _NBLK)
    return out


# speedup vs baseline: 1.1604x; 1.0269x over previous
import functools

import jax
import jax.numpy as jnp
from jax import lax
from jax.experimental import pallas as pl
from jax.experimental.pallas import tpu as pltpu
from jax.experimental.pallas import tpu_sc as plsc

E = 320000
N = 10000
D = 128
H = 32
CO = 65

NC = 2
NS = 16
NW = NC * NS

P = 4
EP = E // P
EPW = EP // NW
CH = 50
OCH = CH // 2
NROWS = EPW // CH
NBUF = 5
NT = NROWS // NBUF


def _gather_add_body(src_hbm, dst_hbm, table_hbm, out_hbm,
                     ibuf,
                     ab0, ab1, ab2, ab3, ab4, bb0, bb1, bb2, bb3, bb4,
                     sb0, sb1, sb2, sb3, sb4,
                     g0, g1, g2, g3, g4, s0, s1, s2, s3, s4):
    abufs = (ab0, ab1, ab2, ab3, ab4)
    bbufs = (bb0, bb1, bb2, bb3, bb4)
    sbufs = (sb0, sb1, sb2, sb3, sb4)
    gsems = (g0, g1, g2, g3, g4)
    ssems = (s0, s1, s2, s3, s4)
    wid = lax.axis_index("s") * NC + lax.axis_index("c")
    obase = wid * (EPW // 2)

    def fetch_idx(t):
        pltpu.sync_copy(src_hbm.at[wid, t], ibuf.at[0])
        pltpu.sync_copy(dst_hbm.at[wid, t], ibuf.at[1])

    def issue(b):
        pltpu.async_copy(table_hbm.at[ibuf.at[0, b]], abufs[b], gsems[b])
        pltpu.async_copy(table_hbm.at[ibuf.at[1, b]], bbufs[b], gsems[b])

    def wait_gather(b):
        pltpu.make_async_copy(
            table_hbm.at[ibuf.at[0, b]], abufs[b], gsems[b]).wait()
        pltpu.make_async_copy(
            table_hbm.at[ibuf.at[1, b]], bbufs[b], gsems[b]).wait()

    def wait_store(cc, b):
        pltpu.make_async_copy(
            sbufs[b], out_hbm.at[pl.ds(obase + cc * OCH, OCH)], ssems[b]).wait()

    fetch_idx(0)
    for b in range(NBUF):
        issue(b)

    def outer(t, carry):
        c0 = t * NBUF
        for b in range(NBUF):
            cc = c0 + b
            wait_gather(b)
            a, bb, sb = abufs[b], bbufs[b], sbufs[b]

            def add_row(r2, carry2, a=a, bb=bb, sb=sb):
                r0 = r2 * 2
                r1 = r0 + 1
                for q in range(D // 16):
                    slc = pl.ds(q * 16, 16)
                    s0v = a[r0, slc] + bb[r0, slc]
                    s1v = a[r1, slc] + bb[r1, slc]
                    v = jnp.stack([s0v.astype(jnp.bfloat16),
                                   s1v.astype(jnp.bfloat16)], axis=0)
                    sb[r2, :, slc] = v
                return carry2
            lax.fori_loop(0, CH // 2, add_row, 0, unroll=2)
            pltpu.async_copy(sb, out_hbm.at[pl.ds(obase + cc * OCH, OCH)],
                             ssems[b])

        @pl.when(t < NT - 1)
        def _prep():
            fetch_idx(t + 1)
            for b in range(NBUF):
                wait_store(c0 + b, b)
                issue(b)
        return carry

    lax.fori_loop(0, NT, outer, 0)
    for b in range(NBUF):
        wait_store(0, b)


@functools.lru_cache(maxsize=None)
def _make_gather_add():
    return pl.kernel(
        _gather_add_body,
        out_type=jax.ShapeDtypeStruct((EP // 2, 2, D), jnp.bfloat16),
        mesh=plsc.VectorSubcoreMesh(core_axis_name="c", subcore_axis_name="s",
                                    num_cores=NC, num_subcores=NS),
        scratch_types=[
            pltpu.VMEM((2, NBUF, CH), jnp.int32),
        ] + [pltpu.VMEM((CH, D), jnp.float32) for _ in range(2 * NBUF)]
          + [pltpu.VMEM((OCH, 2, D), jnp.bfloat16) for _ in range(NBUF)]
          + [pltpu.SemaphoreType.DMA for _ in range(2 * NBUF)],
    )


import numpy as _np
_PACK_PERM = _np.array(
    [32 * g + (j // 2) + 16 * (j % 2) for g in range(4) for j in range(32)])

BLK = 2000


def _mlp_body(e_ref, w1_ref, b1_ref, w2_ref, b2_ref, o_ref):
    e = e_ref[...].astype(jnp.float32)
    h = jnp.dot(e, w1_ref[...], preferred_element_type=jnp.float32)
    h = jnp.maximum(h + b1_ref[...], 0.0)
    o = jnp.dot(h, w2_ref[...], preferred_element_type=jnp.float32)
    o = jnp.maximum(o + b2_ref[...], 0.0)
    m = jnp.max(o, axis=1, keepdims=True)
    p = jnp.exp(o - m)
    o_ref[...] = p / jnp.sum(p, axis=1, keepdims=True)


def _mlp_acc_body(dest_ref, e_ref, w1_ref, b1_ref, w2_ref, b2_ref, o_ref):
    del dest_ref
    _mlp_body(e_ref, w1_ref, b1_ref, w2_ref, b2_ref, o_ref)


_NBLK = EP // BLK


def _mlp_phase(e, W1, b1, W2, b2, dest, block0):
    wspecs = [
        pl.BlockSpec((D, H), lambda i: (0, 0)),
        pl.BlockSpec((1, H), lambda i: (0, 0)),
        pl.BlockSpec((H, CO), lambda i: (0, 0)),
        pl.BlockSpec((1, CO), lambda i: (0, 0)),
    ]
    if dest is None:
        return pl.pallas_call(
            _mlp_body,
            grid=(_NBLK,),
            in_specs=[pl.BlockSpec((BLK, D), lambda i: (i, 0))] + wspecs,
            out_specs=pl.BlockSpec((BLK, CO), lambda i: (i, 0)),
            out_shape=jax.ShapeDtypeStruct((E, CO), jnp.float32),
        )(e, W1, b1, W2, b2)
    return pl.pallas_call(
        _mlp_acc_body,
        grid=(_NBLK,),
        in_specs=[pl.BlockSpec(memory_space=pl.ANY),
                  pl.BlockSpec((BLK, D), lambda i: (i, 0))] + wspecs,
        out_specs=pl.BlockSpec((BLK, CO), lambda i: (i + block0, 0)),
        out_shape=jax.ShapeDtypeStruct((E, CO), jnp.float32),
        input_output_aliases={0: 0},
    )(dest, e, W1, b1, W2, b2)


def kernel(src, dst, table, W1, b1, W2, b2):
    b1r = b1.reshape(1, H)
    b2r = b2.reshape(1, CO)
    gather = _make_gather_add()
    es = []
    for ph in range(P):
        s_p = lax.slice_in_dim(src, ph * EP, (ph + 1) * EP)
        d_p = lax.slice_in_dim(dst, ph * EP, (ph + 1) * EP)
        e2 = gather(s_p.reshape(NW, NT, NBUF, CH),
                    d_p.reshape(NW, NT, NBUF, CH), table)
        es.append(e2.reshape(EP, D))
    out = _mlp_phase(es[0], W1, b1r, W2, b2r, None, 0)
    for ph in range(1, P):
        out = _mlp_phase(es[ph], W1, b1r, W2, b2r, out, ph * _NBLK)
    return out
